# baseline (speedup 1.0000x reference)
import numpy as np
import jax
import jax.numpy as jnp
from jax.experimental import pallas as pl
from jax.experimental.pallas import tpu as pltpu

N = 128
M = 64
HC = 64
NUM_RBF = 32
CUT_UP = 5.0
BA = 64

_START = float(np.exp(-CUT_UP))
_BETA = float((2.0 / NUM_RBF * (1.0 - _START)) ** -2)
_PI_OVER_CUT = float(np.pi / CUT_UP)

_DN_T = (((1,), (1,)), ((), ()))


def _body(pos_ref, extp_ref, extq_ref, w1_ref, b1_ref, w2_ref, b2_ref,
          w3_ref, b3_ref, out_ref):
    E = BA * M

    px = pos_ref[:, 0:1]
    py = pos_ref[:, 1:2]
    pz = pos_ref[:, 2:3]
    ept = jnp.swapaxes(extp_ref[0], 0, 1)
    ex = ept[0:1, :]
    ey = ept[1:2, :]
    ez = ept[2:3, :]
    q = extq_ref[0, 0:1, :]

    vx = px - ex
    vy = py - ey
    vz = pz - ez
    d2 = vx * vx + vy * vy + vz * vz
    d = jnp.sqrt(d2)
    rinv = 1.0 / d
    ux = vx * rinv
    uy = vy * rinv
    uz = vz * rinv

    cut = 0.5 * (jnp.cos(d * _PI_OVER_CUT) + 1.0)
    cut = cut * (d < CUT_UP).astype(jnp.float32)

    riota = jax.lax.broadcasted_iota(jnp.int32, (1, 1, NUM_RBF), 2)
    means = _START + riota.astype(jnp.float32) * ((1.0 - _START) / (NUM_RBF - 1))
    g = jnp.exp(-d)[:, :, None] - means
    rbf = cut[:, :, None] * jnp.exp((-_BETA) * g * g)

    x = rbf.reshape(E, NUM_RBF)
    h = jax.lax.dot_general(x, w1_ref[...], _DN_T,
                            preferred_element_type=jnp.float32) + b1_ref[...]
    h = h * jax.nn.sigmoid(h)
    h = jax.lax.dot_general(h, w2_ref[...], _DN_T,
                            preferred_element_type=jnp.float32) + b2_ref[...]
    h = h * jax.nn.sigmoid(h)
    h3 = jax.lax.dot_general(h, w3_ref[...], _DN_T,
                             preferred_element_type=jnp.float32) + b3_ref[...]
    h3 = (h3 * jax.nn.sigmoid(h3)).reshape(BA, M, 3 * HC)
    a0 = h3[:, :, 0:HC]
    a1 = h3[:, :, HC:2 * HC]
    a2 = h3[:, :, 2 * HC:3 * HC]

    cq = cut * q
    wx = ux * cq
    wy = uy * cq
    wz = uz * cq
    q3 = (ux * ux + uy * uy + uz * uz) * jnp.float32(1.0 / 3.0)
    wxx = ux * wx - q3 * cq
    wyy = uy * wy - q3 * cq
    wzz = uz * wz - q3 * cq
    wxy = ux * wy
    wxz = ux * wz
    wyz = uy * wz

    rc = jax.lax.broadcasted_iota(jnp.int32, (1, 9, 1), 1)

    def rmask(k):
        return (rc == k).astype(jnp.float32)

    def sub(t):
        return t[:, None, :]

    E9 = (rmask(0) + rmask(4) + rmask(8)) * sub(cq)
    K9 = ((rmask(2) - rmask(6)) * sub(wy)
          + (rmask(3) - rmask(1)) * sub(wz)
          + (rmask(7) - rmask(5)) * sub(wx))
    S9 = (rmask(0) * sub(wxx) + rmask(4) * sub(wyy) + rmask(8) * sub(wzz)
          + (rmask(1) + rmask(3)) * sub(wxy)
          + (rmask(2) + rmask(6)) * sub(wxz)
          + (rmask(5) + rmask(7)) * sub(wyz))

    dn = (((1,), (2,)), ((0,), (0,)))
    msg = jax.lax.dot_general(a0, E9, dn, preferred_element_type=jnp.float32)
    msg += jax.lax.dot_general(a1, K9, dn, preferred_element_type=jnp.float32)
    msg += jax.lax.dot_general(a2, S9, dn, preferred_element_type=jnp.float32)
    out_ref[...] = msg


def kernel(pos, ext_pos, ext_charge, batch, W1, b1, W2, b2, W3, b3):
    tot_atoms = pos.shape[0]
    bm = ext_pos.shape[0]
    nb = bm // M

    w3s = W3.reshape(HC, 3, 2 * HC).transpose(1, 0, 2).reshape(3 * HC, 2 * HC)
    b3s = b3.reshape(HC, 3).T.reshape(1, 3 * HC)

    out = pl.pallas_call(
        _body,
        grid=(tot_atoms // BA,),
        in_specs=[
            pl.BlockSpec((BA, 3), lambda i: (i, 0)),
            pl.BlockSpec((1, M, 3), lambda i: (i * BA // N, 0, 0)),
            pl.BlockSpec((1, 1, M), lambda i: (i * BA // N, 0, 0)),
            pl.BlockSpec((HC, NUM_RBF), lambda i: (0, 0)),
            pl.BlockSpec((1, HC), lambda i: (0, 0)),
            pl.BlockSpec((2 * HC, HC), lambda i: (0, 0)),
            pl.BlockSpec((1, 2 * HC), lambda i: (0, 0)),
            pl.BlockSpec((3 * HC, 2 * HC), lambda i: (0, 0)),
            pl.BlockSpec((1, 3 * HC), lambda i: (0, 0)),
        ],
        out_specs=pl.BlockSpec((BA, HC, 9), lambda i: (i, 0, 0)),
        out_shape=jax.ShapeDtypeStruct((tot_atoms, HC, 9), jnp.float32),
        compiler_params=pltpu.CompilerParams(
            dimension_semantics=("parallel",)),
    )(pos, ext_pos.reshape(nb, M, 3), ext_charge.reshape(nb, 1, M),
      W1, b1.reshape(1, HC), W2, b2.reshape(1, 2 * HC), w3s, b3s)
    return out.reshape(tot_atoms, HC, 3, 3)


# speedup vs baseline: 1.2271x; 1.2271x over previous
import numpy as np
import jax
import jax.numpy as jnp
from jax.experimental import pallas as pl
from jax.experimental.pallas import tpu as pltpu

N = 128
M = 64
HC = 64
NUM_RBF = 32
CUT_UP = 5.0
BA = 128

_START = float(np.exp(-CUT_UP))
_BETA = float((2.0 / NUM_RBF * (1.0 - _START)) ** -2)
_PI_OVER_CUT = float(np.pi / CUT_UP)

_DN_T = (((1,), (1,)), ((), ()))


def _body(pos_ref, extp_ref, extq_ref, w1_ref, b1_ref, w2_ref, b2_ref,
          w3_ref, b3_ref, out_ref):
    E = BA * M

    px = pos_ref[:, 0:1]
    py = pos_ref[:, 1:2]
    pz = pos_ref[:, 2:3]
    ept = jnp.swapaxes(extp_ref[0], 0, 1)
    ex = ept[0:1, :]
    ey = ept[1:2, :]
    ez = ept[2:3, :]
    q = extq_ref[0, 0:1, :]

    vx = px - ex
    vy = py - ey
    vz = pz - ez
    d2 = vx * vx + vy * vy + vz * vz
    d = jnp.sqrt(d2)
    rinv = 1.0 / d
    ux = vx * rinv
    uy = vy * rinv
    uz = vz * rinv

    cut = 0.5 * (jnp.cos(d * _PI_OVER_CUT) + 1.0)
    cut = cut * (d < CUT_UP).astype(jnp.float32)

    riota = jax.lax.broadcasted_iota(jnp.int32, (1, 1, NUM_RBF), 2)
    means = _START + riota.astype(jnp.float32) * ((1.0 - _START) / (NUM_RBF - 1))
    g = jnp.exp(-d)[:, :, None] - means
    rbf = cut[:, :, None] * jnp.exp((-_BETA) * g * g)

    x = rbf.reshape(E, NUM_RBF)
    h = jax.lax.dot_general(x, w1_ref[...], _DN_T,
                            preferred_element_type=jnp.float32) + b1_ref[...]
    h = h * jax.nn.sigmoid(h)
    h = jax.lax.dot_general(h, w2_ref[...], _DN_T,
                            preferred_element_type=jnp.float32) + b2_ref[...]
    h = h * jax.nn.sigmoid(h)
    h3 = jax.lax.dot_general(h, w3_ref[...], _DN_T,
                             preferred_element_type=jnp.float32) + b3_ref[...]
    h3 = (h3 * jax.nn.sigmoid(h3)).reshape(BA, M, 3 * HC)
    a0 = h3[:, :, 0:HC]
    a1 = h3[:, :, HC:2 * HC]
    a2 = h3[:, :, 2 * HC:3 * HC]

    cq = cut * q
    wx = ux * cq
    wy = uy * cq
    wz = uz * cq
    q3 = (ux * ux + uy * uy + uz * uz) * jnp.float32(1.0 / 3.0)
    wxx = ux * wx - q3 * cq
    wyy = uy * wy - q3 * cq
    wzz = uz * wz - q3 * cq
    wxy = ux * wy
    wxz = ux * wz
    wyz = uy * wz

    rc = jax.lax.broadcasted_iota(jnp.int32, (1, 9, 1), 1)

    def rmask(k):
        return (rc == k).astype(jnp.float32)

    def sub(t):
        return t[:, None, :]

    E9 = (rmask(0) + rmask(4) + rmask(8)) * sub(cq)
    K9 = ((rmask(2) - rmask(6)) * sub(wy)
          + (rmask(3) - rmask(1)) * sub(wz)
          + (rmask(7) - rmask(5)) * sub(wx))
    S9 = (rmask(0) * sub(wxx) + rmask(4) * sub(wyy) + rmask(8) * sub(wzz)
          + (rmask(1) + rmask(3)) * sub(wxy)
          + (rmask(2) + rmask(6)) * sub(wxz)
          + (rmask(5) + rmask(7)) * sub(wyz))

    dn = (((2,), (1,)), ((0,), (0,)))
    msg = jax.lax.dot_general(E9, a0, dn, preferred_element_type=jnp.float32)
    msg += jax.lax.dot_general(K9, a1, dn, preferred_element_type=jnp.float32)
    msg += jax.lax.dot_general(S9, a2, dn, preferred_element_type=jnp.float32)
    out_ref[...] = msg


def kernel(pos, ext_pos, ext_charge, batch, W1, b1, W2, b2, W3, b3):
    tot_atoms = pos.shape[0]
    bm = ext_pos.shape[0]
    nb = bm // M

    w3s = W3.reshape(HC, 3, 2 * HC).transpose(1, 0, 2).reshape(3 * HC, 2 * HC)
    b3s = b3.reshape(HC, 3).T.reshape(1, 3 * HC)

    out = pl.pallas_call(
        _body,
        grid=(tot_atoms // BA,),
        in_specs=[
            pl.BlockSpec((BA, 3), lambda i: (i, 0)),
            pl.BlockSpec((1, M, 3), lambda i: (i * BA // N, 0, 0)),
            pl.BlockSpec((1, 1, M), lambda i: (i * BA // N, 0, 0)),
            pl.BlockSpec((HC, NUM_RBF), lambda i: (0, 0)),
            pl.BlockSpec((1, HC), lambda i: (0, 0)),
            pl.BlockSpec((2 * HC, HC), lambda i: (0, 0)),
            pl.BlockSpec((1, 2 * HC), lambda i: (0, 0)),
            pl.BlockSpec((3 * HC, 2 * HC), lambda i: (0, 0)),
            pl.BlockSpec((1, 3 * HC), lambda i: (0, 0)),
        ],
        out_specs=pl.BlockSpec((BA, 9, HC), lambda i: (i, 0, 0)),
        out_shape=jax.ShapeDtypeStruct((tot_atoms, 9, HC), jnp.float32),
        compiler_params=pltpu.CompilerParams(
            dimension_semantics=("parallel",)),
    )(pos, ext_pos.reshape(nb, M, 3), ext_charge.reshape(nb, 1, M),
      W1, b1.reshape(1, HC), W2, b2.reshape(1, 2 * HC), w3s, b3s)
    return out.reshape(tot_atoms, 3, 3, HC).transpose(0, 3, 1, 2)


# speedup vs baseline: 1.3026x; 1.0616x over previous
import numpy as np
import jax
import jax.numpy as jnp
from jax.experimental import pallas as pl
from jax.experimental.pallas import tpu as pltpu

N = 128
M = 64
HC = 64
NUM_RBF = 32
CUT_UP = 5.0
BA = 128

_START = float(np.exp(-CUT_UP))
_BETA = float((2.0 / NUM_RBF * (1.0 - _START)) ** -2)
_PI_OVER_CUT = float(np.pi / CUT_UP)

_DN_T = (((1,), (1,)), ((), ()))


def _body(pos_ref, extp_ref, extq_ref, w1_ref, b1_ref, w2_ref, b2_ref,
          w3_ref, b3_ref, out_ref):
    E = BA * M

    px = pos_ref[:, 0:1]
    py = pos_ref[:, 1:2]
    pz = pos_ref[:, 2:3]
    ept = jnp.swapaxes(extp_ref[0], 0, 1)
    ex = ept[0:1, :]
    ey = ept[1:2, :]
    ez = ept[2:3, :]
    q = extq_ref[0, 0:1, :]

    vx = px - ex
    vy = py - ey
    vz = pz - ez
    d2 = vx * vx + vy * vy + vz * vz
    d = jnp.sqrt(d2)
    rinv = 1.0 / d
    ux = vx * rinv
    uy = vy * rinv
    uz = vz * rinv

    cut = 0.5 * (jnp.cos(d * _PI_OVER_CUT) + 1.0)
    cut = cut * (d < CUT_UP).astype(jnp.float32)

    riota = jax.lax.broadcasted_iota(jnp.int32, (1, 1, NUM_RBF), 2)
    means = _START + riota.astype(jnp.float32) * ((1.0 - _START) / (NUM_RBF - 1))
    g = jnp.exp(-d)[:, :, None] - means
    rbf = cut[:, :, None] * jnp.exp((-_BETA) * g * g)

    x = rbf.reshape(E, NUM_RBF)
    h = jnp.dot(x, w1_ref[...], preferred_element_type=jnp.float32) + b1_ref[...]
    h = h * jax.nn.sigmoid(h)
    h = jnp.dot(h, w2_ref[...], preferred_element_type=jnp.float32) + b2_ref[...]
    h = h * jax.nn.sigmoid(h)
    h3 = jnp.dot(h, w3_ref[...], preferred_element_type=jnp.float32) + b3_ref[...]
    h3 = (h3 * jax.nn.sigmoid(h3)).reshape(BA, M, 3 * HC)
    a0 = h3[:, :, 0:HC]
    a1 = h3[:, :, HC:2 * HC]
    a2 = h3[:, :, 2 * HC:3 * HC]

    cq = cut * q
    wx = ux * cq
    wy = uy * cq
    wz = uz * cq
    q3 = (ux * ux + uy * uy + uz * uz) * jnp.float32(1.0 / 3.0)
    wxx = ux * wx - q3 * cq
    wyy = uy * wy - q3 * cq
    wzz = uz * wz - q3 * cq
    wxy = ux * wy
    wxz = ux * wz
    wyz = uy * wz

    rc = jax.lax.broadcasted_iota(jnp.int32, (1, 9, 1), 1)

    def rmask(k):
        return (rc == k).astype(jnp.float32)

    def sub(t):
        return t[:, None, :]

    E9 = (rmask(0) + rmask(4) + rmask(8)) * sub(cq)
    K9 = ((rmask(2) - rmask(6)) * sub(wy)
          + (rmask(3) - rmask(1)) * sub(wz)
          + (rmask(7) - rmask(5)) * sub(wx))
    S9 = (rmask(0) * sub(wxx) + rmask(4) * sub(wyy) + rmask(8) * sub(wzz)
          + (rmask(1) + rmask(3)) * sub(wxy)
          + (rmask(2) + rmask(6)) * sub(wxz)
          + (rmask(5) + rmask(7)) * sub(wyz))

    dn = (((2,), (1,)), ((0,), (0,)))
    msg = jax.lax.dot_general(E9, a0, dn, preferred_element_type=jnp.float32)
    msg += jax.lax.dot_general(K9, a1, dn, preferred_element_type=jnp.float32)
    msg += jax.lax.dot_general(S9, a2, dn, preferred_element_type=jnp.float32)
    out_ref[...] = msg


def kernel(pos, ext_pos, ext_charge, batch, W1, b1, W2, b2, W3, b3):
    tot_atoms = pos.shape[0]
    bm = ext_pos.shape[0]
    nb = bm // M

    w3s = W3.reshape(HC, 3, 2 * HC).transpose(1, 0, 2).reshape(3 * HC, 2 * HC)
    b3s = b3.reshape(HC, 3).T.reshape(1, 3 * HC)

    out = pl.pallas_call(
        _body,
        grid=(tot_atoms // BA,),
        in_specs=[
            pl.BlockSpec((BA, 3), lambda i: (i, 0)),
            pl.BlockSpec((1, M, 3), lambda i: (i * BA // N, 0, 0)),
            pl.BlockSpec((1, 1, M), lambda i: (i * BA // N, 0, 0)),
            pl.BlockSpec((NUM_RBF, HC), lambda i: (0, 0)),
            pl.BlockSpec((1, HC), lambda i: (0, 0)),
            pl.BlockSpec((HC, 2 * HC), lambda i: (0, 0)),
            pl.BlockSpec((1, 2 * HC), lambda i: (0, 0)),
            pl.BlockSpec((2 * HC, 3 * HC), lambda i: (0, 0)),
            pl.BlockSpec((1, 3 * HC), lambda i: (0, 0)),
        ],
        out_specs=pl.BlockSpec((BA, 9, HC), lambda i: (i, 0, 0)),
        out_shape=jax.ShapeDtypeStruct((tot_atoms, 9, HC), jnp.float32),
        compiler_params=pltpu.CompilerParams(
            dimension_semantics=("parallel",)),
    )(pos, ext_pos.reshape(nb, M, 3), ext_charge.reshape(nb, 1, M),
      W1.T, b1.reshape(1, HC), W2.T, b2.reshape(1, 2 * HC), w3s.T, b3s)
    return out.reshape(tot_atoms, 3, 3, HC).transpose(0, 3, 1, 2)
